# baseline (speedup 1.0000x reference)
import functools

import jax
import jax.numpy as jnp
from jax import lax
from jax.experimental import pallas as pl
from jax.experimental.pallas import tpu as pltpu
from jax.experimental.pallas import tpu_sc as plsc

EMBED_DIM = 64
NUM_BINS = 50
VOCAB = NUM_BINS + 3
B = 4096
G = 200

NC = 2
NS = 16
NW = NC * NS
UNITS = G * 8
PER_W = UNITS // NW
BLV = B // 16


def _sc_kernel(idx_hbm, norm_hbm, table_hbm, w_hbm, b_hbm, out_hbm,
               table_v, w_v, b_v, wsplat_v, idx_v, norm_v, out_v):
    wid = lax.axis_index("s") * NC + lax.axis_index("c")

    pltpu.sync_copy(table_hbm, table_v)
    pltpu.sync_copy(w_hbm, w_v)
    pltpu.sync_copy(b_hbm, b_v)

    def fold_b(i, _):
        for j in range(4):
            s = pl.ds(i * EMBED_DIM + j * 16, 16)
            table_v[s] = table_v[s] + b_v[pl.ds(j * 16, 16)]
        return 0
    lax.fori_loop(0, VOCAB, fold_b, 0)

    for j in range(4):
        wj = w_v[pl.ds(16 * j, 16)]
        for l in range(16):
            wsplat_v[pl.ds((16 * j + l) * 16, 16)] = jnp.broadcast_to(
                wj[l], (16,))

    def unit_body(k, _):
        u = wid * PER_W + k
        g = u % G
        dh = u // G
        pltpu.sync_copy(idx_hbm.at[g], idx_v)
        pltpu.sync_copy(norm_hbm.at[g], norm_v)

        dbase = dh * 8
        wds = tuple(wsplat_v[pl.ds((dbase + dl) * 16, 16)] for dl in range(8))
        dh8 = jnp.broadcast_to(dbase, (16,))

        @plsc.parallel_loop(0, BLV)
        def blv_body(i):
            iv = idx_v[pl.ds(i * 16, 16)]
            nv = norm_v[pl.ds(i * 16, 16)]
            base = iv * EMBED_DIM + dh8
            bh = i >> 3
            bl = (i & 7) << 4
            for dl in range(8):
                row = plsc.load_gather(table_v, [base + dl])
                out_v[bh, dl, pl.ds(bl, 16)] = row + nv * wds[dl]

        pltpu.sync_copy(out_v, out_hbm.at[g, dh])
        return 0
    lax.fori_loop(0, PER_W, unit_body, 0)


@jax.jit
def _run(idx, norm, table, w, b):
    mesh = plsc.VectorSubcoreMesh(core_axis_name="c", subcore_axis_name="s")
    kern = functools.partial(
        pl.kernel,
        mesh=mesh,
        compiler_params=pltpu.CompilerParams(needs_layout_passes=False),
        out_type=jax.ShapeDtypeStruct((G, 8, 32, 8, 128), jnp.float32),
        scratch_types=[
            pltpu.VMEM((VOCAB * EMBED_DIM,), jnp.float32),
            pltpu.VMEM((EMBED_DIM,), jnp.float32),
            pltpu.VMEM((EMBED_DIM,), jnp.float32),
            pltpu.VMEM((EMBED_DIM * 16,), jnp.float32),
            pltpu.VMEM((B,), jnp.int32),
            pltpu.VMEM((B,), jnp.float32),
            pltpu.VMEM((32, 8, 128), jnp.float32),
        ],
    )(_sc_kernel)
    out5 = kern(idx, norm, table, w, b)
    return out5.transpose(2, 4, 0, 1, 3).reshape(B, G, EMBED_DIM)


def kernel(discrete_expression, normalized_expr, bin_table, W, b):
    idx = discrete_expression.astype(jnp.int32).T
    norm = normalized_expr.T
    table = bin_table.reshape(-1)
    w = W.reshape(-1)
    return _run(idx, norm, table, w, b)


# speedup vs baseline: 2.9604x; 2.9604x over previous
import functools

import jax
import jax.numpy as jnp
from jax import lax
from jax.experimental import pallas as pl
from jax.experimental.pallas import tpu as pltpu
from jax.experimental.pallas import tpu_sc as plsc

EMBED_DIM = 64
NUM_BINS = 50
VOCAB = NUM_BINS + 3
B = 4096
G = 200

NC = 2
NS = 16
NW = NC * NS
UNITS = G * 8
PER_W = UNITS // NW
BLV = B // 16


TSTRIDE = EMBED_DIM + 1


def _sc_kernel(idx_hbm, norm_hbm, table_hbm, w_hbm, b_hbm, out_hbm,
               table_v, tpad_v, w_v, b_v, wsplat_v, idx_v, norm_v, out_v):
    wid = lax.axis_index("s") * NC + lax.axis_index("c")

    pltpu.sync_copy(table_hbm, table_v)
    pltpu.sync_copy(w_hbm, w_v)
    pltpu.sync_copy(b_hbm, b_v)

    iota = lax.iota(jnp.int32, 16)

    def fold_b(i, _):
        for j in range(4):
            row = (table_v[pl.ds(i * EMBED_DIM + j * 16, 16)]
                   + b_v[pl.ds(j * 16, 16)])
            addr = jnp.broadcast_to(i * TSTRIDE + j * 16, (16,)) + iota
            plsc.store_scatter(tpad_v, [addr], row)
        return 0
    lax.fori_loop(0, VOCAB, fold_b, 0)

    for j in range(4):
        wj = w_v[pl.ds(16 * j, 16)]
        for l in range(16):
            wsplat_v[pl.ds((16 * j + l) * 16, 16)] = jnp.broadcast_to(
                wj[l], (16,))

    def unit_body(k, _):
        u = wid * PER_W + k
        g = u % G
        dh = u // G
        pltpu.sync_copy(idx_hbm.at[g], idx_v)
        pltpu.sync_copy(norm_hbm.at[g], norm_v)

        dbase = dh * 8
        wds = tuple(wsplat_v[pl.ds((dbase + dl) * 16, 16)] for dl in range(8))
        dh8 = jnp.broadcast_to(dbase, (16,))

        @plsc.parallel_loop(0, BLV)
        def blv_body(i):
            iv = idx_v[pl.ds(i * 16, 16)]
            nv = norm_v[pl.ds(i * 16, 16)]
            base = iv * TSTRIDE + dh8
            bh = i >> 3
            bl = (i & 7) << 4
            for dl in range(8):
                row = plsc.load_gather(tpad_v, [base + dl])
                out_v[bh, dl, pl.ds(bl, 16)] = row + nv * wds[dl]

        pltpu.sync_copy(out_v, out_hbm.at[g, dh])
        return 0
    lax.fori_loop(0, PER_W, unit_body, 0)


@jax.jit
def _run(idx, norm, table, w, b):
    mesh = plsc.VectorSubcoreMesh(core_axis_name="c", subcore_axis_name="s")
    kern = functools.partial(
        pl.kernel,
        mesh=mesh,
        compiler_params=pltpu.CompilerParams(needs_layout_passes=False),
        out_type=jax.ShapeDtypeStruct((G, 8, 32, 8, 128), jnp.float32),
        scratch_types=[
            pltpu.VMEM((VOCAB * EMBED_DIM,), jnp.float32),
            pltpu.VMEM((VOCAB * TSTRIDE + 11,), jnp.float32),
            pltpu.VMEM((EMBED_DIM,), jnp.float32),
            pltpu.VMEM((EMBED_DIM,), jnp.float32),
            pltpu.VMEM((EMBED_DIM * 16,), jnp.float32),
            pltpu.VMEM((B,), jnp.int32),
            pltpu.VMEM((B,), jnp.float32),
            pltpu.VMEM((32, 8, 128), jnp.float32),
        ],
    )(_sc_kernel)
    out5 = kern(idx, norm, table, w, b)
    return out5.transpose(2, 4, 0, 1, 3).reshape(B, G, EMBED_DIM)


def kernel(discrete_expression, normalized_expr, bin_table, W, b):
    idx = discrete_expression.astype(jnp.int32).T
    norm = normalized_expr.T
    table = bin_table.reshape(-1)
    w = W.reshape(-1)
    return _run(idx, norm, table, w, b)


# speedup vs baseline: 6.4087x; 2.1648x over previous
import functools

import jax
import jax.numpy as jnp
from jax import lax
from jax.experimental import pallas as pl
from jax.experimental.pallas import tpu as pltpu
from jax.experimental.pallas import tpu_sc as plsc

EMBED_DIM = 64
NUM_BINS = 50
VOCAB = NUM_BINS + 3
B = 4096
G = 200

NC = 2
NS = 16
NW = NC * NS
UNITS = G * 8
PER_W = UNITS // NW
BLV = B // 16


TSTRIDE = EMBED_DIM + 1


def _sc_kernel(idx_hbm, norm_hbm, table_hbm, w_hbm, b_hbm, out_hbm,
               table_v, tpad_v, w_v, b_v, wsplat_v,
               idx_v0, norm_v0, out_v0, idx_v1, norm_v1, out_v1,
               si0, si1, so0, so1):
    wid = lax.axis_index("s") * NC + lax.axis_index("c")

    pltpu.sync_copy(table_hbm, table_v)
    pltpu.sync_copy(w_hbm, w_v)
    pltpu.sync_copy(b_hbm, b_v)

    iota = lax.iota(jnp.int32, 16)

    def fold_b(i, _):
        for j in range(4):
            row = (table_v[pl.ds(i * EMBED_DIM + j * 16, 16)]
                   + b_v[pl.ds(j * 16, 16)])
            addr = jnp.broadcast_to(i * TSTRIDE + j * 16, (16,)) + iota
            plsc.store_scatter(tpad_v, [addr], row)
        return 0
    lax.fori_loop(0, VOCAB, fold_b, 0)

    for j in range(4):
        wj = w_v[pl.ds(16 * j, 16)]
        for l in range(16):
            wsplat_v[pl.ds((16 * j + l) * 16, 16)] = jnp.broadcast_to(
                wj[l], (16,))

    def gdh(k):
        u = wid * PER_W + k
        return u % G, u // G

    def in_start(k, idx_v, norm_v, si):
        g, _ = gdh(k)
        pltpu.async_copy(idx_hbm.at[g], idx_v, si)
        pltpu.async_copy(norm_hbm.at[g], norm_v, si)

    def in_drain(idx_v, norm_v, si):
        pltpu.make_async_copy(idx_hbm.at[0], idx_v, si).wait()
        pltpu.make_async_copy(norm_hbm.at[0], norm_v, si).wait()

    def out_drain(out_v, so):
        pltpu.make_async_copy(out_hbm.at[0, 0], out_v, so).wait()

    def compute(k, idx_v, norm_v, out_v):
        _, dh = gdh(k)
        dbase = dh * 8
        wds = tuple(wsplat_v[pl.ds((dbase + dl) * 16, 16)] for dl in range(8))
        dh8 = jnp.broadcast_to(dbase, (16,))

        @plsc.parallel_loop(0, BLV)
        def blv_body(i):
            iv = idx_v[pl.ds(i * 16, 16)]
            nv = norm_v[pl.ds(i * 16, 16)]
            base = iv * TSTRIDE + dh8
            bh = i >> 3
            bl = (i & 7) << 4
            for dl in range(8):
                row = plsc.load_gather(tpad_v, [base + dl])
                out_v[bh, dl, pl.ds(bl, 16)] = row + nv * wds[dl]

    def out_start(k, out_v, so):
        g, dh = gdh(k)
        pltpu.async_copy(out_v, out_hbm.at[g, dh], so)

    in_start(0, idx_v0, norm_v0, si0)

    def pipe_body(kk, _):
        kA = 2 * kk
        kB = kA + 1

        in_start(kB, idx_v1, norm_v1, si1)
        in_drain(idx_v0, norm_v0, si0)

        @pl.when(kk > 0)
        def _():
            out_drain(out_v0, so0)
        compute(kA, idx_v0, norm_v0, out_v0)
        out_start(kA, out_v0, so0)

        @pl.when(kk < PER_W // 2 - 1)
        def _():
            in_start(kA + 2, idx_v0, norm_v0, si0)
        in_drain(idx_v1, norm_v1, si1)

        @pl.when(kk > 0)
        def _():
            out_drain(out_v1, so1)
        compute(kB, idx_v1, norm_v1, out_v1)
        out_start(kB, out_v1, so1)
        return 0

    lax.fori_loop(0, PER_W // 2, pipe_body, 0)
    out_drain(out_v0, so0)
    out_drain(out_v1, so1)


@jax.jit
def _run(idx, norm, table, w, b):
    mesh = plsc.VectorSubcoreMesh(core_axis_name="c", subcore_axis_name="s")
    kern = functools.partial(
        pl.kernel,
        mesh=mesh,
        compiler_params=pltpu.CompilerParams(needs_layout_passes=False),
        out_type=jax.ShapeDtypeStruct((G, 8, 32, 8, 128), jnp.float32),
        scratch_types=[
            pltpu.VMEM((VOCAB * EMBED_DIM,), jnp.float32),
            pltpu.VMEM((VOCAB * TSTRIDE + 11,), jnp.float32),
            pltpu.VMEM((EMBED_DIM,), jnp.float32),
            pltpu.VMEM((EMBED_DIM,), jnp.float32),
            pltpu.VMEM((EMBED_DIM * 16,), jnp.float32),
            pltpu.VMEM((B,), jnp.int32),
            pltpu.VMEM((B,), jnp.float32),
            pltpu.VMEM((32, 8, 128), jnp.float32),
            pltpu.VMEM((B,), jnp.int32),
            pltpu.VMEM((B,), jnp.float32),
            pltpu.VMEM((32, 8, 128), jnp.float32),
            pltpu.SemaphoreType.DMA,
            pltpu.SemaphoreType.DMA,
            pltpu.SemaphoreType.DMA,
            pltpu.SemaphoreType.DMA,
        ],
    )(_sc_kernel)
    out5 = kern(idx, norm, table, w, b)
    return out5.transpose(2, 4, 0, 1, 3).reshape(B, G, EMBED_DIM)


def kernel(discrete_expression, normalized_expr, bin_table, W, b):
    idx = discrete_expression.astype(jnp.int32).T
    norm = normalized_expr.T
    table = bin_table.reshape(-1)
    w = W.reshape(-1)
    return _run(idx, norm, table, w, b)


# speedup vs baseline: 6.4276x; 1.0029x over previous
import functools

import jax
import jax.numpy as jnp
from jax import lax
from jax.experimental import pallas as pl
from jax.experimental.pallas import tpu as pltpu
from jax.experimental.pallas import tpu_sc as plsc

EMBED_DIM = 64
NUM_BINS = 50
VOCAB = NUM_BINS + 3
B = 4096
G = 200

NC = 2
NS = 16
NW = NC * NS
UNITS = G * 8
PER_W = UNITS // NW
BLV = B // 16


TSTRIDE = EMBED_DIM + 1


def _sc_kernel(idx_hbm, norm_hbm, table_hbm, w_hbm, b_hbm, out_hbm,
               table_v, tpad_v, w_v, b_v, wsplat_v,
               idx_v0, norm_v0, out_v0, idx_v1, norm_v1, out_v1,
               si0, si1, so0, so1):
    wid = lax.axis_index("s") * NC + lax.axis_index("c")

    g0 = (wid * PER_W) % G
    pltpu.async_copy(idx_hbm.at[g0], idx_v0, si0)
    pltpu.async_copy(norm_hbm.at[g0], norm_v0, si0)

    pltpu.sync_copy(table_hbm, table_v)
    pltpu.sync_copy(w_hbm, w_v)
    pltpu.sync_copy(b_hbm, b_v)

    iota = lax.iota(jnp.int32, 16)

    def fold_b(i, _):
        for j in range(4):
            row = (table_v[pl.ds(i * EMBED_DIM + j * 16, 16)]
                   + b_v[pl.ds(j * 16, 16)])
            addr = jnp.broadcast_to(i * TSTRIDE + j * 16, (16,)) + iota
            plsc.store_scatter(tpad_v, [addr], row)
        return 0
    lax.fori_loop(0, VOCAB, fold_b, 0)

    for j in range(4):
        wj = w_v[pl.ds(16 * j, 16)]
        for l in range(16):
            wsplat_v[pl.ds((16 * j + l) * 16, 16)] = jnp.broadcast_to(
                wj[l], (16,))

    def gdh(k):
        u = wid * PER_W + k
        return u % G, u // G

    def in_start(k, idx_v, norm_v, si):
        g, _ = gdh(k)
        pltpu.async_copy(idx_hbm.at[g], idx_v, si)
        pltpu.async_copy(norm_hbm.at[g], norm_v, si)

    def in_drain(idx_v, norm_v, si):
        pltpu.make_async_copy(idx_hbm.at[0], idx_v, si).wait()
        pltpu.make_async_copy(norm_hbm.at[0], norm_v, si).wait()

    def out_drain(out_v, so):
        pltpu.make_async_copy(out_hbm.at[0, 0], out_v, so).wait()

    def compute(k, idx_v, norm_v, out_v):
        _, dh = gdh(k)
        dbase = dh * 8
        wds = tuple(wsplat_v[pl.ds((dbase + dl) * 16, 16)] for dl in range(8))
        dh8 = jnp.broadcast_to(dbase, (16,))

        @plsc.parallel_loop(0, BLV)
        def blv_body(i):
            iv = idx_v[pl.ds(i * 16, 16)]
            nv = norm_v[pl.ds(i * 16, 16)]
            base = iv * TSTRIDE + dh8
            bh = i >> 3
            bl = (i & 7) << 4
            for dl in range(8):
                row = plsc.load_gather(tpad_v, [base + dl])
                out_v[bh, dl, pl.ds(bl, 16)] = row + nv * wds[dl]

    def out_start(k, out_v, so):
        g, dh = gdh(k)
        pltpu.async_copy(out_v, out_hbm.at[g, dh], so)

    def pipe_body(kk, _):
        kA = 2 * kk
        kB = kA + 1

        in_start(kB, idx_v1, norm_v1, si1)
        in_drain(idx_v0, norm_v0, si0)

        @pl.when(kk > 0)
        def _():
            out_drain(out_v0, so0)
        compute(kA, idx_v0, norm_v0, out_v0)
        out_start(kA, out_v0, so0)

        @pl.when(kk < PER_W // 2 - 1)
        def _():
            in_start(kA + 2, idx_v0, norm_v0, si0)
        in_drain(idx_v1, norm_v1, si1)

        @pl.when(kk > 0)
        def _():
            out_drain(out_v1, so1)
        compute(kB, idx_v1, norm_v1, out_v1)
        out_start(kB, out_v1, so1)
        return 0

    lax.fori_loop(0, PER_W // 2, pipe_body, 0)
    out_drain(out_v0, so0)
    out_drain(out_v1, so1)


@jax.jit
def _run(idx, norm, table, w, b):
    mesh = plsc.VectorSubcoreMesh(core_axis_name="c", subcore_axis_name="s")
    kern = functools.partial(
        pl.kernel,
        mesh=mesh,
        compiler_params=pltpu.CompilerParams(needs_layout_passes=False),
        out_type=jax.ShapeDtypeStruct((G, 8, 32, 8, 128), jnp.float32),
        scratch_types=[
            pltpu.VMEM((VOCAB * EMBED_DIM,), jnp.float32),
            pltpu.VMEM((VOCAB * TSTRIDE + 11,), jnp.float32),
            pltpu.VMEM((EMBED_DIM,), jnp.float32),
            pltpu.VMEM((EMBED_DIM,), jnp.float32),
            pltpu.VMEM((EMBED_DIM * 16,), jnp.float32),
            pltpu.VMEM((B,), jnp.int32),
            pltpu.VMEM((B,), jnp.float32),
            pltpu.VMEM((32, 8, 128), jnp.float32),
            pltpu.VMEM((B,), jnp.int32),
            pltpu.VMEM((B,), jnp.float32),
            pltpu.VMEM((32, 8, 128), jnp.float32),
            pltpu.SemaphoreType.DMA,
            pltpu.SemaphoreType.DMA,
            pltpu.SemaphoreType.DMA,
            pltpu.SemaphoreType.DMA,
        ],
    )(_sc_kernel)
    out5 = kern(idx, norm, table, w, b)
    return out5.transpose(2, 4, 0, 1, 3).reshape(B, G, EMBED_DIM)


def kernel(discrete_expression, normalized_expr, bin_table, W, b):
    idx = discrete_expression.astype(jnp.int32).T
    norm = normalized_expr.T
    table = bin_table.reshape(-1)
    w = W.reshape(-1)
    return _run(idx, norm, table, w, b)
